# baseline (speedup 1.0000x reference)
import functools

import jax
import jax.numpy as jnp
from jax import lax
from jax.experimental import pallas as pl
from jax.experimental.pallas import tpu as pltpu
from jax.experimental.pallas import tpu_sc as plsc

HEADS1, OUT1 = 8, 8
HEADS2, OUT2 = 1, 16

NC, NS, NW = 2, 16, 32


def _mesh():
    return plsc.VectorSubcoreMesh(core_axis_name="c", subcore_axis_name="s",
                                  num_cores=NC, num_subcores=NS)


def _pick_chunk(ew, align, maxk=2000):
    for k in (2000, 1600, 1000, 800, 640, 512, 400, 320, 256, 200, 160, 128):
        if k <= maxk and ew % k == 0 and k % align == 0:
            return k
    return ew


def _dense1_body(x_ref, w_ref, as_ref, ad_ref, h_ref, att_ref, exl_ref):
    h = jnp.dot(x_ref[...], w_ref[...], preferred_element_type=jnp.float32)
    h_ref[...] = h
    a_s = jnp.dot(h, as_ref[...], preferred_element_type=jnp.float32)
    a_d = jnp.dot(h, ad_ref[...], preferred_element_type=jnp.float32)
    att_ref[...] = jnp.concatenate([a_s, a_d], axis=1)
    e = a_s + a_d
    exl_ref[...] = jnp.exp(jnp.maximum(e, 0.2 * e))


def _dense1(x, W1, As_mat, Ad_mat):
    N = x.shape[0]
    return pl.pallas_call(
        _dense1_body,
        out_shape=(jax.ShapeDtypeStruct((N, HEADS1 * OUT1), jnp.float32),
                   jax.ShapeDtypeStruct((N, 2 * HEADS1), jnp.float32),
                   jax.ShapeDtypeStruct((N, HEADS1), jnp.float32)),
    )(x, W1, As_mat, Ad_mat)


def _recip1_body(dp_ref, exl_ref, out_ref):
    N = exl_ref.shape[0]
    den = dp_ref[0, :N] + dp_ref[1, :N] + exl_ref[...] + 1e-16
    out_ref[...] = 1.0 / den


def _recip1(dp, exl1):
    N = exl1.shape[0]
    return pl.pallas_call(
        _recip1_body,
        out_shape=jax.ShapeDtypeStruct((N, HEADS1), jnp.float32),
    )(dp, exl1)


def _dense2_body(qp_ref, h1_ref, exl1_ref, rec1_ref, b1_ref, w2_ref,
                 as2_ref, ad2_ref, rep_ref,
                 h2_ref, as2o_ref, ad2o_ref, exl2_ref):
    alpha_loop = exl1_ref[...] * rec1_ref[...]
    loop_term = h1_ref[...] * jnp.dot(alpha_loop, rep_ref[...],
                                      preferred_element_type=jnp.float32)
    N = h1_ref.shape[0]
    out1 = qp_ref[0, :N] + qp_ref[1, :N] + loop_term + b1_ref[...][None, :]
    x2 = jnp.where(out1 > 0, out1, jnp.exp(jnp.minimum(out1, 0.0)) - 1.0)
    h2 = jnp.dot(x2, w2_ref[...], preferred_element_type=jnp.float32)
    h2_ref[...] = h2
    a_s = jnp.dot(h2, as2_ref[...], preferred_element_type=jnp.float32)
    a_d = jnp.dot(h2, ad2_ref[...], preferred_element_type=jnp.float32)
    as2o_ref[...] = a_s
    ad2o_ref[...] = a_d
    e = a_s + a_d
    exl2_ref[...] = jnp.exp(jnp.maximum(e, 0.2 * e))


def _dense2(qp, h1, exl1, rec1, b1, W2, as2v, ad2v, REP):
    N = h1.shape[0]
    return pl.pallas_call(
        _dense2_body,
        out_shape=(jax.ShapeDtypeStruct((N, OUT2), jnp.float32),
                   jax.ShapeDtypeStruct((N, 1), jnp.float32),
                   jax.ShapeDtypeStruct((N, 1), jnp.float32),
                   jax.ShapeDtypeStruct((N, 1), jnp.float32)),
    )(qp, h1, exl1, rec1, b1, W2, as2v, ad2v, REP)


def _recip2_body(dp2_ref, exl2_ref, out_ref):
    den = jnp.sum(dp2_ref[...], axis=0) + exl2_ref[...] + 1e-16
    out_ref[...] = 1.0 / den


def _recip2(dp2, exl2):
    N = exl2.shape[0]
    return pl.pallas_call(
        _recip2_body,
        out_shape=jax.ShapeDtypeStruct((N,), jnp.float32),
    )(dp2, exl2)


def _final_body(op_ref, h2_ref, exl2_ref, rec2_ref, b2_ref, out_ref):
    alpha_loop = exl2_ref[...] * rec2_ref[...]
    N = h2_ref.shape[0]
    y = (op_ref[0, :N] + op_ref[1, :N] + h2_ref[...] * alpha_loop
         + b2_ref[...][None, :])
    m = jnp.max(y, axis=1, keepdims=True)
    s = jnp.log(jnp.sum(jnp.exp(y - m), axis=1, keepdims=True))
    out_ref[...] = y - m - s


def _final(op, h2, exl2c, rec2c, b2):
    N = h2.shape[0]
    return pl.pallas_call(
        _final_body,
        out_shape=jax.ShapeDtypeStruct((N, OUT2), jnp.float32),
    )(op, h2, exl2c, rec2c, b2)


def _sc_att1(att1, src, dst, z8):
    N = att1.shape[0]
    NPAD = z8.shape[0]
    E = src.shape[0]
    EW = E // NW
    K = _pick_chunk(EW, 8, maxk=1000)
    CH = EW // K
    RPT = NPAD // NS

    assert CH % 2 == 0

    @functools.partial(
        pl.kernel,
        out_type=(jax.ShapeDtypeStruct((NC, NPAD, HEADS1), jnp.float32),
                  jax.ShapeDtypeStruct((E, HEADS1), jnp.float32)),
        mesh=_mesh(),
        compiler_params=pltpu.CompilerParams(use_tc_tiling_on_sc=False,
                                             needs_layout_passes=False),
        scratch_types=[
            pltpu.VMEM((2, K), jnp.int32),
            pltpu.VMEM((2, K), jnp.int32),
            pltpu.VMEM((K, 2 * HEADS1), jnp.float32),
            pltpu.VMEM((K, 2 * HEADS1), jnp.float32),
            pltpu.VMEM((K, 2 * HEADS1), jnp.float32),
            pltpu.VMEM((K, 2 * HEADS1), jnp.float32),
            pltpu.VMEM((K, HEADS1), jnp.float32),
            pltpu.VMEM_SHARED((NPAD, HEADS1), jnp.float32),
            pltpu.SemaphoreType.DMA,
            pltpu.SemaphoreType.DMA,
            pltpu.SemaphoreType.DMA,
            pltpu.SemaphoreType.DMA,
        ],
    )
    def k(att_hbm, src_hbm, dst_hbm, z8_hbm, dp_hbm, ex_hbm,
          srcb, dstb, sbuf0, dbuf0, sbuf1, dbuf1, exb, den_sh,
          ss0, sd0, ss1, sd1):
        cid = lax.axis_index("c")
        sid = lax.axis_index("s")
        wid = sid * NC + cid
        base = wid * EW
        r0 = sid * RPT
        sbufs = (sbuf0, sbuf1)
        dbufs = (dbuf0, dbuf1)
        ssems = (ss0, ss1)
        dsems = (sd0, sd1)

        def prefetch(c, p):
            eb = base + c * K
            pltpu.sync_copy(src_hbm.at[pl.ds(eb, K)], srcb.at[p])
            pltpu.sync_copy(dst_hbm.at[pl.ds(eb, K)], dstb.at[p])
            pltpu.async_copy(att_hbm.at[srcb.at[p]], sbufs[p], ssems[p])
            pltpu.async_copy(att_hbm.at[dstb.at[p]], dbufs[p], dsems[p])

        prefetch(0, 0)
        pltpu.sync_copy(z8_hbm.at[pl.ds(r0, RPT)], den_sh.at[pl.ds(r0, RPT)])
        plsc.subcore_barrier()

        lane = lax.iota(jnp.int32, 16)
        pairrow = lane >> 3
        col = lane & 7

        def compute(c, p):
            eb = base + c * K
            pltpu.make_async_copy(att_hbm.at[srcb.at[p]], sbufs[p],
                                  ssems[p]).wait()
            pltpu.make_async_copy(att_hbm.at[dstb.at[p]], dbufs[p],
                                  dsems[p]).wait()
            sbuf = sbufs[p]
            dbuf = dbufs[p]

            @plsc.parallel_loop(0, K // 2, 1, unroll=4)
            def pair(j):
                rv = pairrow + j * 2
                sv = plsc.load_gather(sbuf, [rv, col])
                dv = plsc.load_gather(dbuf, [rv, col + 8])
                ev = sv + dv
                ev = jnp.maximum(ev, 0.2 * ev)
                xv = jnp.exp(ev)
                plsc.store_scatter(exb, [rv, col], xv)
            pltpu.sync_copy(exb, ex_hbm.at[pl.ds(eb, K)])
            pltpu.sync_copy(exb, den_sh.at[dstb.at[p]], add=True)

        def chunk2(cc, _):
            c = cc * 2
            prefetch(c + 1, 1)
            compute(c, 0)

            @pl.when(c + 2 < CH)
            def _():
                prefetch(c + 2, 0)

            compute(c + 1, 1)
            return 0

        lax.fori_loop(0, CH // 2, chunk2, 0)
        plsc.subcore_barrier()
        pltpu.sync_copy(den_sh.at[pl.ds(r0, RPT)],
                        dp_hbm.at[cid, pl.ds(r0, RPT)])

    return k(att1, src, dst, z8)


def _sc_msg1(exbuf, dp1, exl1p, h1, src, dst, z64):
    N = h1.shape[0]
    E = src.shape[0]
    C = HEADS1 * OUT1
    NPAD = z64.shape[0]
    EW = E // NW
    K = _pick_chunk(EW, 8, maxk=1000)
    H0 = ((K // 2 + 7) // 8) * 8
    H1 = K - H0
    CH = EW // K
    RPT = NPAD // NS
    assert CH % 2 == 0 and H1 > 0
    assert K >= 960 and 320 < RPT <= 640 and RPT % 2 == 0

    @functools.partial(
        pl.kernel,
        out_type=(jax.ShapeDtypeStruct((NC, NPAD, C), jnp.float32),
                  jax.ShapeDtypeStruct((NPAD, HEADS1), jnp.float32)),
        mesh=_mesh(),
        compiler_params=pltpu.CompilerParams(use_tc_tiling_on_sc=False,
                                             needs_layout_passes=False),
        scratch_types=[
            pltpu.VMEM((2, K), jnp.int32),
            pltpu.VMEM((2, K), jnp.int32),
            pltpu.VMEM((K, HEADS1), jnp.float32),
            pltpu.VMEM((K, HEADS1), jnp.float32),
            pltpu.VMEM((H0, C), jnp.float32),
            pltpu.VMEM((H1, C), jnp.float32),
            pltpu.VMEM_SHARED((NPAD, C), jnp.float32),
            pltpu.SemaphoreType.DMA,
            pltpu.SemaphoreType.DMA,
            pltpu.SemaphoreType.DMA,
        ],
    )
    def k(ex_hbm, dp_hbm, exl_hbm, h_hbm, src_hbm, dst_hbm, z64_hbm,
          out_hbm, rec_hbm, srcc, dstc, recb, exb, hb0, hb1, out_sh,
          srec, sh0, sh1):
        cid = lax.axis_index("c")
        sid = lax.axis_index("s")
        wid = sid * NC + cid
        base = wid * EW
        r0 = sid * RPT

        lane0 = lax.iota(jnp.int32, 16)
        pairrow0 = lane0 >> 3
        col0 = lane0 & 7

        for off, ln in ((0, 320), (320, RPT - 320)):
            pltpu.sync_copy(dp_hbm.at[0, pl.ds(r0 + off, ln)],
                            exb.at[pl.ds(0, ln)])
            pltpu.sync_copy(dp_hbm.at[1, pl.ds(r0 + off, ln)],
                            exb.at[pl.ds(320, ln)])
            pltpu.sync_copy(exl_hbm.at[pl.ds(r0 + off, ln)],
                            exb.at[pl.ds(640, ln)])

            @plsc.parallel_loop(0, ln // 2, 1, unroll=4)
            def rp(j, off=off):
                rv = pairrow0 + j * 2
                a = plsc.load_gather(exb, [rv, col0])
                b = plsc.load_gather(exb, [rv + 320, col0])
                cv = plsc.load_gather(exb, [rv + 640, col0])
                r = 1.0 / (a + b + cv + 1e-16)
                plsc.store_scatter(recb, [rv + off, col0], r)

        pltpu.sync_copy(recb.at[pl.ds(0, RPT)], rec_hbm.at[pl.ds(r0, RPT)])

        pltpu.sync_copy(src_hbm.at[pl.ds(base, K)], srcc.at[0])
        pltpu.sync_copy(dst_hbm.at[pl.ds(base, K)], dstc.at[0])
        pltpu.sync_copy(z64_hbm.at[pl.ds(r0, RPT)], out_sh.at[pl.ds(r0, RPT)])
        plsc.subcore_barrier()
        pltpu.async_copy(rec_hbm.at[dstc.at[0]], recb, srec)
        pltpu.async_copy(h_hbm.at[srcc.at[0, pl.ds(0, H0)]], hb0, sh0)
        pltpu.async_copy(h_hbm.at[srcc.at[0, pl.ds(H0, H1)]], hb1, sh1)

        lane = lax.iota(jnp.int32, 16)
        pairrow = lane >> 3
        col = lane & 7

        def do_chunk(c, px):
            qx = 1 - px
            pltpu.sync_copy(ex_hbm.at[pl.ds(base + c * K, K)], exb)
            pltpu.make_async_copy(rec_hbm.at[dstc.at[px]], recb, srec).wait()

            @plsc.parallel_loop(0, K // 2, 1, unroll=8)
            def pair(j):
                rv = pairrow + j * 2
                exv = plsc.load_gather(exb, [rv, col])
                rcv = plsc.load_gather(recb, [rv, col])
                plsc.store_scatter(exb, [rv, col], exv * rcv)

            @pl.when(c + 1 < CH)
            def _():
                nb = base + (c + 1) * K
                pltpu.sync_copy(src_hbm.at[pl.ds(nb, K)], srcc.at[qx])
                pltpu.sync_copy(dst_hbm.at[pl.ds(nb, K)], dstc.at[qx])
                pltpu.async_copy(rec_hbm.at[dstc.at[qx]], recb, srec)

            for aoff, hlen, hb, sh in ((0, H0, hb0, sh0), (H0, H1, hb1, sh1)):
                pltpu.make_async_copy(
                    h_hbm.at[srcc.at[px, pl.ds(aoff, hlen)]], hb, sh).wait()

                @plsc.parallel_loop(0, hlen, 1, unroll=8)
                def edge(e):
                    eb16 = jnp.zeros((16,), jnp.int32) + (e + aoff)
                    for v in range(C // 16):
                        hv = hb[e, pl.ds(v * 16, 16)]
                        al = plsc.load_gather(exb, [eb16, pairrow + v * 2])
                        hb[e, pl.ds(v * 16, 16)] = hv * al
                pltpu.sync_copy(hb, out_sh.at[dstc.at[px, pl.ds(aoff, hlen)]],
                                add=True)

                @pl.when(c + 1 < CH)
                def _():
                    pltpu.async_copy(
                        h_hbm.at[srcc.at[qx, pl.ds(aoff, hlen)]], hb, sh)

        def chunk2(cc, _):
            do_chunk(cc * 2, 0)
            do_chunk(cc * 2 + 1, 1)
            return 0

        lax.fori_loop(0, CH // 2, chunk2, 0)
        plsc.subcore_barrier()
        pltpu.sync_copy(out_sh.at[pl.ds(r0, RPT)],
                        out_hbm.at[cid, pl.ds(r0, RPT)])

    return k(exbuf, dp1, exl1p, h1, src, dst, z64)


def _sc_att2(as2, ad2, src, dst, z1):
    N = as2.shape[0]
    NPAD = ((N + 8 * NS - 1) // (8 * NS)) * (8 * NS)
    E = src.shape[0]
    EW = E // NW
    K = _pick_chunk(EW, 16)
    CH = EW // K

    @functools.partial(
        pl.kernel,
        out_type=(jax.ShapeDtypeStruct(((NW - 1) * N + NPAD,), jnp.float32),
                  jax.ShapeDtypeStruct((E,), jnp.float32)),
        mesh=_mesh(),
        compiler_params=pltpu.CompilerParams(use_tc_tiling_on_sc=False,
                                             needs_layout_passes=False),
        scratch_types=[
            pltpu.VMEM((N,), jnp.float32),
            pltpu.VMEM((N,), jnp.float32),
            pltpu.VMEM((N,), jnp.float32),
            pltpu.VMEM((K,), jnp.int32),
            pltpu.VMEM((K,), jnp.int32),
            pltpu.VMEM((K,), jnp.float32),
        ],
    )
    def k(as_hbm, ad_hbm, src_hbm, dst_hbm, z1_hbm, dp_hbm, ex_hbm,
          asb, adb, denb, srcb, dstb, exb):
        cid = lax.axis_index("c")
        sid = lax.axis_index("s")
        wid = sid * NC + cid
        base = wid * EW
        pltpu.sync_copy(as_hbm, asb)
        pltpu.sync_copy(ad_hbm, adb)
        pltpu.sync_copy(z1_hbm, denb)

        def chunk(c, _):
            eb = base + c * K
            pltpu.sync_copy(src_hbm.at[pl.ds(eb, K)], srcb)
            pltpu.sync_copy(dst_hbm.at[pl.ds(eb, K)], dstb)

            @plsc.parallel_loop(0, K // 16, 1, unroll=4)
            def grp(j):
                srcv = srcb[pl.ds(j * 16, 16)]
                dstv = dstb[pl.ds(j * 16, 16)]
                sv = plsc.load_gather(asb, [srcv])
                dv = plsc.load_gather(adb, [dstv])
                ev = sv + dv
                ev = jnp.maximum(ev, 0.2 * ev)
                xv = jnp.exp(ev)
                exb[pl.ds(j * 16, 16)] = xv
                plsc.addupdate_scatter(denb, [dstv], xv)
            pltpu.sync_copy(exb, ex_hbm.at[pl.ds(eb, K)])
            return 0

        lax.fori_loop(0, CH, chunk, 0)
        pltpu.sync_copy(denb, dp_hbm.at[pl.ds(wid * N, N)])

    return k(as2, ad2, src, dst, z1)


def _sc_msg2(exbuf2, dp2, exl2p, h2, src, dst, z16):
    N = h2.shape[0]
    E = src.shape[0]
    C = OUT2
    NPAD = z16.shape[0]
    EW = E // NW
    K = _pick_chunk(EW, 8, maxk=1000)
    KP = ((K + 15) // 16) * 16
    CH = EW // K
    RPT = NPAD // NS

    assert CH % 2 == 0

    @functools.partial(
        pl.kernel,
        out_type=(jax.ShapeDtypeStruct((NC, NPAD, C), jnp.float32),
                  jax.ShapeDtypeStruct((NPAD,), jnp.float32)),
        mesh=_mesh(),
        compiler_params=pltpu.CompilerParams(use_tc_tiling_on_sc=False,
                                             needs_layout_passes=False),
        scratch_types=[
            pltpu.VMEM((N,), jnp.float32),
            pltpu.VMEM((NW, 640), jnp.float32),
            pltpu.VMEM((2, K), jnp.int32),
            pltpu.VMEM((2, KP), jnp.int32),
            pltpu.VMEM((KP,), jnp.float32),
            pltpu.VMEM((K, C), jnp.float32),
            pltpu.VMEM((K, C), jnp.float32),
            pltpu.VMEM_SHARED((NPAD, C), jnp.float32),
            pltpu.SemaphoreType.DMA,
            pltpu.SemaphoreType.DMA,
            pltpu.SemaphoreType.DMA,
        ],
    )
    def k(ex_hbm, dp_hbm, exl_hbm, h_hbm, src_hbm, dst_hbm, z16_hbm,
          out_hbm, rec_hbm, recb, pbuf, srcb, dstb, exb, hb0, hb1, out_sh,
          sh0, sh1, sp):
        cid = lax.axis_index("c")
        sid = lax.axis_index("s")
        wid = sid * NC + cid
        base = wid * EW
        r0 = sid * RPT
        hbs = (hb0, hb1)
        hsems = (sh0, sh1)
        NV = (RPT + 15) // 16

        pltpu.sync_copy(exl_hbm.at[pl.ds(r0, RPT)], recb.at[pl.ds(0, RPT)])
        for w in range(NW):
            pltpu.async_copy(dp_hbm.at[pl.ds(w * N + r0, RPT)],
                             pbuf.at[w, pl.ds(0, RPT)], sp)
        for w in range(NW):
            pltpu.make_async_copy(dp_hbm.at[pl.ds(w * N + r0, RPT)],
                                  pbuf.at[w, pl.ds(0, RPT)], sp).wait()

        @plsc.parallel_loop(0, NV, 1, unroll=2)
        def rp(i):
            den = recb[pl.ds(i * 16, 16)] + 1e-16
            for w in range(NW):
                den = den + pbuf[w, pl.ds(i * 16, 16)]
            recb[pl.ds(640 + i * 16, 16)] = 1.0 / den

        pltpu.sync_copy(recb.at[pl.ds(640, RPT)], rec_hbm.at[pl.ds(r0, RPT)])

        def prefetch(c, p):
            eb = base + c * K
            pltpu.sync_copy(src_hbm.at[pl.ds(eb, K)], srcb.at[p])
            pltpu.sync_copy(dst_hbm.at[pl.ds(eb, K)], dstb.at[p, pl.ds(0, K)])
            pltpu.async_copy(h_hbm.at[srcb.at[p]], hbs[p], hsems[p])

        if KP > K:
            zi = jnp.zeros((16,), jnp.int32)
            dstb[0, pl.ds(KP - 16, 16)] = zi
            dstb[1, pl.ds(KP - 16, 16)] = zi
            exb[pl.ds(KP - 16, 16)] = jnp.zeros((16,), jnp.float32)
        pltpu.sync_copy(z16_hbm.at[pl.ds(r0, RPT)], out_sh.at[pl.ds(r0, RPT)])
        plsc.subcore_barrier()
        pltpu.sync_copy(rec_hbm.at[pl.ds(0, N)], recb)
        prefetch(0, 0)

        def compute(c, p):
            eb = base + c * K
            pltpu.sync_copy(ex_hbm.at[pl.ds(eb, K)], exb.at[pl.ds(0, K)])

            @plsc.parallel_loop(0, KP // 16, 1, unroll=4)
            def grp(j):
                dstv = dstb[p, pl.ds(j * 16, 16)]
                rv = plsc.load_gather(recb, [dstv])
                exv = exb[pl.ds(j * 16, 16)]
                exb[pl.ds(j * 16, 16)] = exv * rv
            pltpu.make_async_copy(h_hbm.at[srcb.at[p]], hbs[p],
                                  hsems[p]).wait()
            hb = hbs[p]

            @plsc.parallel_loop(0, K, 1, unroll=4)
            def edge(e):
                hv = hb[e, :]
                al = plsc.load_gather(exb, [jnp.zeros((16,), jnp.int32) + e])
                hb[e, :] = hv * al
            pltpu.sync_copy(hb, out_sh.at[dstb.at[p, pl.ds(0, K)]], add=True)

        def chunk2(cc, _):
            c = cc * 2
            prefetch(c + 1, 1)
            compute(c, 0)

            @pl.when(c + 2 < CH)
            def _():
                prefetch(c + 2, 0)

            compute(c + 1, 1)
            return 0

        lax.fori_loop(0, CH // 2, chunk2, 0)
        plsc.subcore_barrier()
        pltpu.sync_copy(out_sh.at[pl.ds(r0, RPT)],
                        out_hbm.at[cid, pl.ds(r0, RPT)])

    return k(exbuf2, dp2, exl2p, h2, src, dst, z16)


def kernel(train_data, train_edge_index, W1, att_src1, att_dst1, b1,
           W2, att_src2, att_dst2, b2):
    x = train_data
    N = x.shape[0]
    src = train_edge_index[0]
    dst = train_edge_index[1]

    eye8 = jnp.eye(HEADS1, dtype=jnp.float32)
    As_mat = (att_src1[:, :, None] * eye8[:, None, :]).reshape(
        HEADS1 * OUT1, HEADS1)
    Ad_mat = (att_dst1[:, :, None] * eye8[:, None, :]).reshape(
        HEADS1 * OUT1, HEADS1)
    REP = jnp.broadcast_to(eye8[:, :, None],
                           (HEADS1, HEADS1, OUT1)).reshape(HEADS1,
                                                           HEADS1 * OUT1)
    as2v = att_src2.reshape(HEADS2 * OUT2, 1)
    ad2v = att_dst2.reshape(HEADS2 * OUT2, 1)

    npad = ((N + 8 * NS - 1) // (8 * NS)) * (8 * NS)
    z64 = jnp.zeros((npad, HEADS1 * OUT1), jnp.float32)
    z8 = jnp.zeros((npad, HEADS1), jnp.float32)
    z16 = jnp.zeros((npad, OUT2), jnp.float32)
    z1 = jnp.zeros((N,), jnp.float32)

    h1, att1, exl1 = _dense1(x, W1, As_mat, Ad_mat)
    dp1, exbuf1 = _sc_att1(att1, src, dst, z8)
    exl1p = jnp.pad(exl1, ((0, npad - N), (0, 0)))
    qp1, rec1p = _sc_msg1(exbuf1, dp1, exl1p, h1, src, dst, z64)

    h2, as2c, ad2c, exl2c = _dense2(qp1, h1, exl1, rec1p[:N], b1, W2,
                                    as2v, ad2v, REP)
    as2 = as2c.reshape(N)
    ad2 = ad2c.reshape(N)
    exl2 = exl2c.reshape(N)

    dp2, exbuf2 = _sc_att2(as2, ad2, src, dst, z1)
    exl2p = jnp.pad(exl2, (0, npad - N))
    op2, rec2p = _sc_msg2(exbuf2, dp2, exl2p, h2, src, dst, z16)

    return _final(op2, h2, exl2c, rec2p[:N].reshape(N, 1), b2)


# speedup vs baseline: 1.0055x; 1.0055x over previous
import functools

import jax
import jax.numpy as jnp
from jax import lax
from jax.experimental import pallas as pl
from jax.experimental.pallas import tpu as pltpu
from jax.experimental.pallas import tpu_sc as plsc

HEADS1, OUT1 = 8, 8
HEADS2, OUT2 = 1, 16

NC, NS, NW = 2, 16, 32


def _mesh():
    return plsc.VectorSubcoreMesh(core_axis_name="c", subcore_axis_name="s",
                                  num_cores=NC, num_subcores=NS)


def _pick_chunk(ew, align, maxk=2000):
    for k in (2000, 1600, 1000, 800, 640, 512, 400, 320, 256, 200, 160, 128):
        if k <= maxk and ew % k == 0 and k % align == 0:
            return k
    return ew


def _dense1_body(x_ref, w_ref, as_ref, ad_ref, h_ref, att_ref, exl_ref):
    h = jnp.dot(x_ref[...], w_ref[...], preferred_element_type=jnp.float32)
    h_ref[...] = h
    a_s = jnp.dot(h, as_ref[...], preferred_element_type=jnp.float32)
    a_d = jnp.dot(h, ad_ref[...], preferred_element_type=jnp.float32)
    att_ref[...] = jnp.concatenate([a_s, a_d], axis=1)
    e = a_s + a_d
    exl_ref[...] = jnp.exp(jnp.maximum(e, 0.2 * e))


def _dense1(x, W1, As_mat, Ad_mat):
    N = x.shape[0]
    return pl.pallas_call(
        _dense1_body,
        out_shape=(jax.ShapeDtypeStruct((N, HEADS1 * OUT1), jnp.float32),
                   jax.ShapeDtypeStruct((N, 2 * HEADS1), jnp.float32),
                   jax.ShapeDtypeStruct((N, HEADS1), jnp.float32)),
    )(x, W1, As_mat, Ad_mat)


def _recip1_body(dp_ref, exl_ref, out_ref):
    N = exl_ref.shape[0]
    den = dp_ref[0, :N] + dp_ref[1, :N] + exl_ref[...] + 1e-16
    out_ref[...] = 1.0 / den


def _recip1(dp, exl1):
    N = exl1.shape[0]
    return pl.pallas_call(
        _recip1_body,
        out_shape=jax.ShapeDtypeStruct((N, HEADS1), jnp.float32),
    )(dp, exl1)


def _dense2_body(qp_ref, h1_ref, exl1_ref, rec1_ref, b1_ref, w2_ref,
                 as2_ref, ad2_ref, rep_ref,
                 h2_ref, as2o_ref, ad2o_ref, exl2_ref):
    alpha_loop = exl1_ref[...] * rec1_ref[...]
    loop_term = h1_ref[...] * jnp.dot(alpha_loop, rep_ref[...],
                                      preferred_element_type=jnp.float32)
    N = h1_ref.shape[0]
    out1 = qp_ref[0, :N] + qp_ref[1, :N] + loop_term + b1_ref[...][None, :]
    x2 = jnp.where(out1 > 0, out1, jnp.exp(jnp.minimum(out1, 0.0)) - 1.0)
    h2 = jnp.dot(x2, w2_ref[...], preferred_element_type=jnp.float32)
    h2_ref[...] = h2
    a_s = jnp.dot(h2, as2_ref[...], preferred_element_type=jnp.float32)
    a_d = jnp.dot(h2, ad2_ref[...], preferred_element_type=jnp.float32)
    as2o_ref[...] = a_s
    ad2o_ref[...] = a_d
    e = a_s + a_d
    exl2_ref[...] = jnp.exp(jnp.maximum(e, 0.2 * e))


def _dense2(qp, h1, exl1, rec1, b1, W2, as2v, ad2v, REP):
    N = h1.shape[0]
    return pl.pallas_call(
        _dense2_body,
        out_shape=(jax.ShapeDtypeStruct((N, OUT2), jnp.float32),
                   jax.ShapeDtypeStruct((N, 1), jnp.float32),
                   jax.ShapeDtypeStruct((N, 1), jnp.float32),
                   jax.ShapeDtypeStruct((N, 1), jnp.float32)),
    )(qp, h1, exl1, rec1, b1, W2, as2v, ad2v, REP)


def _recip2_body(dp2_ref, exl2_ref, out_ref):
    den = jnp.sum(dp2_ref[...], axis=0) + exl2_ref[...] + 1e-16
    out_ref[...] = 1.0 / den


def _recip2(dp2, exl2):
    N = exl2.shape[0]
    return pl.pallas_call(
        _recip2_body,
        out_shape=jax.ShapeDtypeStruct((N,), jnp.float32),
    )(dp2, exl2)


def _final_body(op_ref, h2_ref, exl2_ref, rec2_ref, b2_ref, out_ref):
    alpha_loop = exl2_ref[...] * rec2_ref[...]
    N = h2_ref.shape[0]
    y = (op_ref[0, :N] + op_ref[1, :N] + h2_ref[...] * alpha_loop
         + b2_ref[...][None, :])
    m = jnp.max(y, axis=1, keepdims=True)
    s = jnp.log(jnp.sum(jnp.exp(y - m), axis=1, keepdims=True))
    out_ref[...] = y - m - s


def _final(op, h2, exl2c, rec2c, b2):
    N = h2.shape[0]
    return pl.pallas_call(
        _final_body,
        out_shape=jax.ShapeDtypeStruct((N, OUT2), jnp.float32),
    )(op, h2, exl2c, rec2c, b2)


def _sc_att1(att1, src, dst, z8):
    N = att1.shape[0]
    NPAD = z8.shape[0]
    E = src.shape[0]
    EW = E // NW
    K = _pick_chunk(EW, 8, maxk=1000)
    CH = EW // K
    RPT = NPAD // NS

    assert CH % 2 == 0

    @functools.partial(
        pl.kernel,
        out_type=(jax.ShapeDtypeStruct((NC, NPAD, HEADS1), jnp.float32),
                  jax.ShapeDtypeStruct((E, HEADS1), jnp.float32)),
        mesh=_mesh(),
        compiler_params=pltpu.CompilerParams(use_tc_tiling_on_sc=False,
                                             needs_layout_passes=False),
        scratch_types=[
            pltpu.VMEM((2, K), jnp.int32),
            pltpu.VMEM((2, K), jnp.int32),
            pltpu.VMEM((K, 2 * HEADS1), jnp.float32),
            pltpu.VMEM((K, 2 * HEADS1), jnp.float32),
            pltpu.VMEM((K, 2 * HEADS1), jnp.float32),
            pltpu.VMEM((K, 2 * HEADS1), jnp.float32),
            pltpu.VMEM((K, HEADS1), jnp.float32),
            pltpu.VMEM_SHARED((NPAD, HEADS1), jnp.float32),
            pltpu.SemaphoreType.DMA,
            pltpu.SemaphoreType.DMA,
            pltpu.SemaphoreType.DMA,
            pltpu.SemaphoreType.DMA,
        ],
    )
    def k(att_hbm, src_hbm, dst_hbm, z8_hbm, dp_hbm, ex_hbm,
          srcb, dstb, sbuf0, dbuf0, sbuf1, dbuf1, exb, den_sh,
          ss0, sd0, ss1, sd1):
        cid = lax.axis_index("c")
        sid = lax.axis_index("s")
        wid = sid * NC + cid
        base = wid * EW
        r0 = sid * RPT
        sbufs = (sbuf0, sbuf1)
        dbufs = (dbuf0, dbuf1)
        ssems = (ss0, ss1)
        dsems = (sd0, sd1)

        def prefetch(c, p):
            eb = base + c * K
            pltpu.sync_copy(src_hbm.at[pl.ds(eb, K)], srcb.at[p])
            pltpu.sync_copy(dst_hbm.at[pl.ds(eb, K)], dstb.at[p])
            pltpu.async_copy(att_hbm.at[srcb.at[p]], sbufs[p], ssems[p])
            pltpu.async_copy(att_hbm.at[dstb.at[p]], dbufs[p], dsems[p])

        prefetch(0, 0)
        pltpu.sync_copy(z8_hbm.at[pl.ds(r0, RPT)], den_sh.at[pl.ds(r0, RPT)])
        plsc.subcore_barrier()

        lane = lax.iota(jnp.int32, 16)
        pairrow = lane >> 3
        col = lane & 7

        def compute(c, p):
            eb = base + c * K
            pltpu.make_async_copy(att_hbm.at[srcb.at[p]], sbufs[p],
                                  ssems[p]).wait()
            pltpu.make_async_copy(att_hbm.at[dstb.at[p]], dbufs[p],
                                  dsems[p]).wait()
            sbuf = sbufs[p]
            dbuf = dbufs[p]

            @plsc.parallel_loop(0, K // 2, 1, unroll=4)
            def pair(j):
                rv = pairrow + j * 2
                sv = plsc.load_gather(sbuf, [rv, col])
                dv = plsc.load_gather(dbuf, [rv, col + 8])
                ev = sv + dv
                ev = jnp.maximum(ev, 0.2 * ev)
                xv = jnp.exp(ev)
                plsc.store_scatter(exb, [rv, col], xv)
            pltpu.sync_copy(exb, ex_hbm.at[pl.ds(eb, K)])
            pltpu.sync_copy(exb, den_sh.at[dstb.at[p]], add=True)

        def chunk2(cc, _):
            c = cc * 2
            prefetch(c + 1, 1)
            compute(c, 0)

            @pl.when(c + 2 < CH)
            def _():
                prefetch(c + 2, 0)

            compute(c + 1, 1)
            return 0

        lax.fori_loop(0, CH // 2, chunk2, 0)
        plsc.subcore_barrier()
        pltpu.sync_copy(den_sh.at[pl.ds(r0, RPT)],
                        dp_hbm.at[cid, pl.ds(r0, RPT)])

    return k(att1, src, dst, z8)


def _sc_msg1(exbuf, dp1, exl1p, h1, src, dst, z64):
    N = h1.shape[0]
    E = src.shape[0]
    C = HEADS1 * OUT1
    NPAD = z64.shape[0]
    EW = E // NW
    K = _pick_chunk(EW, 8, maxk=1000)
    H0 = ((K // 2 + 7) // 8) * 8
    H1 = K - H0
    CH = EW // K
    RPT = NPAD // NS
    assert CH % 2 == 0 and H1 > 0
    assert K >= 960 and 320 < RPT <= 640 and RPT % 2 == 0

    @functools.partial(
        pl.kernel,
        out_type=(jax.ShapeDtypeStruct((NC, NPAD, C), jnp.float32),
                  jax.ShapeDtypeStruct((NPAD, HEADS1), jnp.float32)),
        mesh=_mesh(),
        compiler_params=pltpu.CompilerParams(use_tc_tiling_on_sc=False,
                                             needs_layout_passes=False),
        scratch_types=[
            pltpu.VMEM((2, K), jnp.int32),
            pltpu.VMEM((2, K), jnp.int32),
            pltpu.VMEM((K, HEADS1), jnp.float32),
            pltpu.VMEM((K, HEADS1), jnp.float32),
            pltpu.VMEM((H0, C), jnp.float32),
            pltpu.VMEM((H1, C), jnp.float32),
            pltpu.VMEM_SHARED((NPAD, C), jnp.float32),
            pltpu.SemaphoreType.DMA,
            pltpu.SemaphoreType.DMA,
            pltpu.SemaphoreType.DMA,
        ],
    )
    def k(ex_hbm, dp_hbm, exl_hbm, h_hbm, src_hbm, dst_hbm, z64_hbm,
          out_hbm, rec_hbm, srcc, dstc, recb, exb, hb0, hb1, out_sh,
          srec, sh0, sh1):
        cid = lax.axis_index("c")
        sid = lax.axis_index("s")
        wid = sid * NC + cid
        base = wid * EW
        r0 = sid * RPT

        lane0 = lax.iota(jnp.int32, 16)
        pairrow0 = lane0 >> 3
        col0 = lane0 & 7

        for off, ln in ((0, 320), (320, RPT - 320)):
            pltpu.sync_copy(dp_hbm.at[0, pl.ds(r0 + off, ln)],
                            exb.at[pl.ds(0, ln)])
            pltpu.sync_copy(dp_hbm.at[1, pl.ds(r0 + off, ln)],
                            exb.at[pl.ds(320, ln)])
            pltpu.sync_copy(exl_hbm.at[pl.ds(r0 + off, ln)],
                            exb.at[pl.ds(640, ln)])

            @plsc.parallel_loop(0, ln // 2, 1, unroll=4)
            def rp(j, off=off):
                rv = pairrow0 + j * 2
                a = plsc.load_gather(exb, [rv, col0])
                b = plsc.load_gather(exb, [rv + 320, col0])
                cv = plsc.load_gather(exb, [rv + 640, col0])
                r = 1.0 / (a + b + cv + 1e-16)
                plsc.store_scatter(recb, [rv + off, col0], r)

        pltpu.sync_copy(recb.at[pl.ds(0, RPT)], rec_hbm.at[pl.ds(r0, RPT)])

        pltpu.sync_copy(src_hbm.at[pl.ds(base, K)], srcc.at[0])
        pltpu.sync_copy(dst_hbm.at[pl.ds(base, K)], dstc.at[0])
        pltpu.sync_copy(z64_hbm.at[pl.ds(r0, RPT)], out_sh.at[pl.ds(r0, RPT)])
        plsc.subcore_barrier()
        pltpu.async_copy(rec_hbm.at[dstc.at[0]], recb, srec)
        pltpu.async_copy(h_hbm.at[srcc.at[0, pl.ds(0, H0)]], hb0, sh0)
        pltpu.async_copy(h_hbm.at[srcc.at[0, pl.ds(H0, H1)]], hb1, sh1)

        lane = lax.iota(jnp.int32, 16)
        pairrow = lane >> 3
        col = lane & 7

        def do_chunk(c, px):
            qx = 1 - px
            pltpu.sync_copy(ex_hbm.at[pl.ds(base + c * K, K)], exb)
            pltpu.make_async_copy(rec_hbm.at[dstc.at[px]], recb, srec).wait()

            @plsc.parallel_loop(0, K // 2, 1, unroll=4)
            def pair(j):
                rv = pairrow + j * 2
                exv = plsc.load_gather(exb, [rv, col])
                rcv = plsc.load_gather(recb, [rv, col])
                plsc.store_scatter(exb, [rv, col], exv * rcv)

            @pl.when(c + 1 < CH)
            def _():
                nb = base + (c + 1) * K
                pltpu.sync_copy(src_hbm.at[pl.ds(nb, K)], srcc.at[qx])
                pltpu.sync_copy(dst_hbm.at[pl.ds(nb, K)], dstc.at[qx])
                pltpu.async_copy(rec_hbm.at[dstc.at[qx]], recb, srec)

            for aoff, hlen, hb, sh in ((0, H0, hb0, sh0), (H0, H1, hb1, sh1)):
                pltpu.make_async_copy(
                    h_hbm.at[srcc.at[px, pl.ds(aoff, hlen)]], hb, sh).wait()

                @plsc.parallel_loop(0, hlen, 1, unroll=4)
                def edge(e):
                    eb16 = jnp.zeros((16,), jnp.int32) + (e + aoff)
                    for v in range(C // 16):
                        hv = hb[e, pl.ds(v * 16, 16)]
                        al = plsc.load_gather(exb, [eb16, pairrow + v * 2])
                        hb[e, pl.ds(v * 16, 16)] = hv * al
                pltpu.sync_copy(hb, out_sh.at[dstc.at[px, pl.ds(aoff, hlen)]],
                                add=True)

                @pl.when(c + 1 < CH)
                def _():
                    pltpu.async_copy(
                        h_hbm.at[srcc.at[qx, pl.ds(aoff, hlen)]], hb, sh)

        def chunk2(cc, _):
            do_chunk(cc * 2, 0)
            do_chunk(cc * 2 + 1, 1)
            return 0

        lax.fori_loop(0, CH // 2, chunk2, 0)
        plsc.subcore_barrier()
        pltpu.sync_copy(out_sh.at[pl.ds(r0, RPT)],
                        out_hbm.at[cid, pl.ds(r0, RPT)])

    return k(exbuf, dp1, exl1p, h1, src, dst, z64)


def _sc_att2(as2, ad2, src, dst, z1):
    N = as2.shape[0]
    NPAD = ((N + 8 * NS - 1) // (8 * NS)) * (8 * NS)
    E = src.shape[0]
    EW = E // NW
    K = _pick_chunk(EW, 16)
    CH = EW // K

    @functools.partial(
        pl.kernel,
        out_type=(jax.ShapeDtypeStruct(((NW - 1) * N + NPAD,), jnp.float32),
                  jax.ShapeDtypeStruct((E,), jnp.float32)),
        mesh=_mesh(),
        compiler_params=pltpu.CompilerParams(use_tc_tiling_on_sc=False,
                                             needs_layout_passes=False),
        scratch_types=[
            pltpu.VMEM((N,), jnp.float32),
            pltpu.VMEM((N,), jnp.float32),
            pltpu.VMEM((N,), jnp.float32),
            pltpu.VMEM((K,), jnp.int32),
            pltpu.VMEM((K,), jnp.int32),
            pltpu.VMEM((K,), jnp.float32),
        ],
    )
    def k(as_hbm, ad_hbm, src_hbm, dst_hbm, z1_hbm, dp_hbm, ex_hbm,
          asb, adb, denb, srcb, dstb, exb):
        cid = lax.axis_index("c")
        sid = lax.axis_index("s")
        wid = sid * NC + cid
        base = wid * EW
        pltpu.sync_copy(as_hbm, asb)
        pltpu.sync_copy(ad_hbm, adb)
        pltpu.sync_copy(z1_hbm, denb)

        def chunk(c, _):
            eb = base + c * K
            pltpu.sync_copy(src_hbm.at[pl.ds(eb, K)], srcb)
            pltpu.sync_copy(dst_hbm.at[pl.ds(eb, K)], dstb)

            @plsc.parallel_loop(0, K // 16, 1, unroll=4)
            def grp(j):
                srcv = srcb[pl.ds(j * 16, 16)]
                dstv = dstb[pl.ds(j * 16, 16)]
                sv = plsc.load_gather(asb, [srcv])
                dv = plsc.load_gather(adb, [dstv])
                ev = sv + dv
                ev = jnp.maximum(ev, 0.2 * ev)
                xv = jnp.exp(ev)
                exb[pl.ds(j * 16, 16)] = xv
                plsc.addupdate_scatter(denb, [dstv], xv)
            pltpu.sync_copy(exb, ex_hbm.at[pl.ds(eb, K)])
            return 0

        lax.fori_loop(0, CH, chunk, 0)
        pltpu.sync_copy(denb, dp_hbm.at[pl.ds(wid * N, N)])

    return k(as2, ad2, src, dst, z1)


def _sc_msg2(exbuf2, dp2, exl2p, h2, src, dst, z16):
    N = h2.shape[0]
    E = src.shape[0]
    C = OUT2
    NPAD = z16.shape[0]
    EW = E // NW
    K = _pick_chunk(EW, 8, maxk=1000)
    KP = ((K + 15) // 16) * 16
    CH = EW // K
    RPT = NPAD // NS

    assert CH % 2 == 0

    @functools.partial(
        pl.kernel,
        out_type=(jax.ShapeDtypeStruct((NC, NPAD, C), jnp.float32),
                  jax.ShapeDtypeStruct((NPAD,), jnp.float32)),
        mesh=_mesh(),
        compiler_params=pltpu.CompilerParams(use_tc_tiling_on_sc=False,
                                             needs_layout_passes=False),
        scratch_types=[
            pltpu.VMEM((N,), jnp.float32),
            pltpu.VMEM((NW, 640), jnp.float32),
            pltpu.VMEM((2, K), jnp.int32),
            pltpu.VMEM((2, KP), jnp.int32),
            pltpu.VMEM((KP,), jnp.float32),
            pltpu.VMEM((K, C), jnp.float32),
            pltpu.VMEM((K, C), jnp.float32),
            pltpu.VMEM_SHARED((NPAD, C), jnp.float32),
            pltpu.SemaphoreType.DMA,
            pltpu.SemaphoreType.DMA,
            pltpu.SemaphoreType.DMA,
        ],
    )
    def k(ex_hbm, dp_hbm, exl_hbm, h_hbm, src_hbm, dst_hbm, z16_hbm,
          out_hbm, rec_hbm, recb, pbuf, srcb, dstb, exb, hb0, hb1, out_sh,
          sh0, sh1, sp):
        cid = lax.axis_index("c")
        sid = lax.axis_index("s")
        wid = sid * NC + cid
        base = wid * EW
        r0 = sid * RPT
        hbs = (hb0, hb1)
        hsems = (sh0, sh1)
        NV = (RPT + 15) // 16

        pltpu.sync_copy(exl_hbm.at[pl.ds(r0, RPT)], recb.at[pl.ds(0, RPT)])
        for w in range(NW):
            pltpu.async_copy(dp_hbm.at[pl.ds(w * N + r0, RPT)],
                             pbuf.at[w, pl.ds(0, RPT)], sp)
        for w in range(NW):
            pltpu.make_async_copy(dp_hbm.at[pl.ds(w * N + r0, RPT)],
                                  pbuf.at[w, pl.ds(0, RPT)], sp).wait()

        @plsc.parallel_loop(0, NV, 1, unroll=2)
        def rp(i):
            den = recb[pl.ds(i * 16, 16)] + 1e-16
            for w in range(NW):
                den = den + pbuf[w, pl.ds(i * 16, 16)]
            recb[pl.ds(640 + i * 16, 16)] = 1.0 / den

        pltpu.sync_copy(recb.at[pl.ds(640, RPT)], rec_hbm.at[pl.ds(r0, RPT)])

        def prefetch(c, p):
            eb = base + c * K
            pltpu.sync_copy(src_hbm.at[pl.ds(eb, K)], srcb.at[p])
            pltpu.sync_copy(dst_hbm.at[pl.ds(eb, K)], dstb.at[p, pl.ds(0, K)])
            pltpu.async_copy(h_hbm.at[srcb.at[p]], hbs[p], hsems[p])

        if KP > K:
            zi = jnp.zeros((16,), jnp.int32)
            dstb[0, pl.ds(KP - 16, 16)] = zi
            dstb[1, pl.ds(KP - 16, 16)] = zi
            exb[pl.ds(KP - 16, 16)] = jnp.zeros((16,), jnp.float32)
        pltpu.sync_copy(z16_hbm.at[pl.ds(r0, RPT)], out_sh.at[pl.ds(r0, RPT)])
        plsc.subcore_barrier()
        pltpu.sync_copy(rec_hbm.at[pl.ds(0, N)], recb)
        prefetch(0, 0)

        def compute(c, p):
            eb = base + c * K
            pltpu.sync_copy(ex_hbm.at[pl.ds(eb, K)], exb.at[pl.ds(0, K)])

            @plsc.parallel_loop(0, KP // 16, 1, unroll=4)
            def grp(j):
                dstv = dstb[p, pl.ds(j * 16, 16)]
                rv = plsc.load_gather(recb, [dstv])
                exv = exb[pl.ds(j * 16, 16)]
                exb[pl.ds(j * 16, 16)] = exv * rv
            pltpu.make_async_copy(h_hbm.at[srcb.at[p]], hbs[p],
                                  hsems[p]).wait()
            hb = hbs[p]

            @plsc.parallel_loop(0, K, 1, unroll=4)
            def edge(e):
                hv = hb[e, :]
                al = plsc.load_gather(exb, [jnp.zeros((16,), jnp.int32) + e])
                hb[e, :] = hv * al
            pltpu.sync_copy(hb, out_sh.at[dstb.at[p, pl.ds(0, K)]], add=True)

        def chunk2(cc, _):
            c = cc * 2
            prefetch(c + 1, 1)
            compute(c, 0)

            @pl.when(c + 2 < CH)
            def _():
                prefetch(c + 2, 0)

            compute(c + 1, 1)
            return 0

        lax.fori_loop(0, CH // 2, chunk2, 0)
        plsc.subcore_barrier()
        pltpu.sync_copy(out_sh.at[pl.ds(r0, RPT)],
                        out_hbm.at[cid, pl.ds(r0, RPT)])

    return k(exbuf2, dp2, exl2p, h2, src, dst, z16)


def kernel(train_data, train_edge_index, W1, att_src1, att_dst1, b1,
           W2, att_src2, att_dst2, b2):
    x = train_data
    N = x.shape[0]
    src = train_edge_index[0]
    dst = train_edge_index[1]

    eye8 = jnp.eye(HEADS1, dtype=jnp.float32)
    As_mat = (att_src1[:, :, None] * eye8[:, None, :]).reshape(
        HEADS1 * OUT1, HEADS1)
    Ad_mat = (att_dst1[:, :, None] * eye8[:, None, :]).reshape(
        HEADS1 * OUT1, HEADS1)
    REP = jnp.broadcast_to(eye8[:, :, None],
                           (HEADS1, HEADS1, OUT1)).reshape(HEADS1,
                                                           HEADS1 * OUT1)
    as2v = att_src2.reshape(HEADS2 * OUT2, 1)
    ad2v = att_dst2.reshape(HEADS2 * OUT2, 1)

    npad = ((N + 8 * NS - 1) // (8 * NS)) * (8 * NS)
    z64 = jnp.zeros((npad, HEADS1 * OUT1), jnp.float32)
    z8 = jnp.zeros((npad, HEADS1), jnp.float32)
    z16 = jnp.zeros((npad, OUT2), jnp.float32)
    z1 = jnp.zeros((N,), jnp.float32)

    h1, att1, exl1 = _dense1(x, W1, As_mat, Ad_mat)
    dp1, exbuf1 = _sc_att1(att1, src, dst, z8)
    exl1p = jnp.pad(exl1, ((0, npad - N), (0, 0)))
    qp1, rec1p = _sc_msg1(exbuf1, dp1, exl1p, h1, src, dst, z64)

    h2, as2c, ad2c, exl2c = _dense2(qp1, h1, exl1, rec1p[:N], b1, W2,
                                    as2v, ad2v, REP)
    as2 = as2c.reshape(N)
    ad2 = ad2c.reshape(N)
    exl2 = exl2c.reshape(N)

    dp2, exbuf2 = _sc_att2(as2, ad2, src, dst, z1)
    exl2p = jnp.pad(exl2, (0, npad - N))
    op2, rec2p = _sc_msg2(exbuf2, dp2, exl2p, h2, src, dst, z16)

    return _final(op2, h2, exl2c, rec2p[:N].reshape(N, 1), b2)
